# baseline (speedup 1.0000x reference)
import numpy as np
import jax
import jax.numpy as jnp
from jax.experimental import pallas as pl
from jax.experimental.pallas import tpu as pltpu

_SLAB = 8


def _masked_idx(c: int, ratio: float) -> list:
    mask = np.linspace(0, c * (1 - ratio), int(c * ratio))
    return [i for i in range(c) if i in mask]


def _make_dma_kernel(c, cb, nmask):
    def _dma_kernel(x_ref, n_ref, o_ref, nz_vmem, *rest):
        slabs = rest[:nmask]
        sems = rest[nmask:]
        s = iter(sems)
        big_sems = [next(s) for _ in range(nmask)]
        nz_sem = next(s)
        slab_in_sems = [next(s) for _ in range(nmask)]
        slab_out_sems = [next(s) for _ in range(nmask)]

        big = [
            pltpu.make_async_copy(
                x_ref.at[:, pl.ds(j * cb + _SLAB, cb - _SLAB), :],
                o_ref.at[:, pl.ds(j * cb + _SLAB, cb - _SLAB), :],
                big_sems[j],
            )
            for j in range(nmask)
        ]
        nz_cp = pltpu.make_async_copy(n_ref, nz_vmem, nz_sem)
        slab_in = [
            pltpu.make_async_copy(
                x_ref.at[:, pl.ds(j * cb, _SLAB), :],
                slabs[j],
                slab_in_sems[j],
            )
            for j in range(nmask)
        ]
        for cp in big + [nz_cp] + slab_in:
            cp.start()
        nz_cp.wait()
        slab_out = []
        for j in range(nmask):
            slab_in[j].wait()
            x = slabs[j][...]
            row = jax.lax.broadcasted_iota(jnp.int32, x.shape, 1)
            slabs[j][...] = jnp.where(row == 0, nz_vmem[j][:, None, :], x)
            cp = pltpu.make_async_copy(
                slabs[j],
                o_ref.at[:, pl.ds(j * cb, _SLAB), :],
                slab_out_sems[j],
            )
            cp.start()
            slab_out.append(cp)
        for cp in slab_out + big:
            cp.wait()

    return _dma_kernel


def kernel(input1, noise):
    b, c, h, w = input1.shape
    hw = h * w
    idx = _masked_idx(c, 0.5)
    nmask = len(idx)
    cb = c // nmask
    if idx != [j * cb for j in range(nmask)]:
        raise ValueError("masked channels not uniformly spaced")

    x = input1.reshape(b, c, hw)
    out = pl.pallas_call(
        _make_dma_kernel(c, cb, nmask),
        in_specs=[
            pl.BlockSpec(memory_space=pl.ANY),
            pl.BlockSpec(memory_space=pl.ANY),
        ],
        out_specs=pl.BlockSpec(memory_space=pl.ANY),
        out_shape=jax.ShapeDtypeStruct((b, c, hw), x.dtype),
        scratch_shapes=(
            [pltpu.VMEM((nmask, b, hw), jnp.float32)]
            + [pltpu.VMEM((b, _SLAB, hw), jnp.float32) for _ in range(nmask)]
            + [pltpu.SemaphoreType.DMA] * (3 * nmask + 1)
        ),
    )(x, noise)
    return out.reshape(b, c, h, w)


# speedup vs baseline: 11.6385x; 11.6385x over previous
import numpy as np
import jax
import jax.numpy as jnp
from jax.experimental import pallas as pl


def _masked_idx(c: int, ratio: float) -> list:
    mask = np.linspace(0, c * (1 - ratio), int(c * ratio))
    return [i for i in range(c) if i in mask]


def _copy_mask_kernel(x_ref, n_ref, o_ref):
    x = x_ref[0]
    nz = n_ref[0, 0]
    row = jax.lax.broadcasted_iota(jnp.int32, x.shape, 0)
    o_ref[0] = jnp.where(row == 0, nz, x)


def kernel(input1, noise):
    b, c, h, w = input1.shape
    hw = h * w
    idx = _masked_idx(c, 0.5)
    nmask = len(idx)
    cb = c // nmask
    if idx != [j * cb for j in range(nmask)]:
        raise ValueError("masked channels not uniformly spaced")

    x = input1.reshape(b, c, hw)
    nz = noise.reshape(nmask, b, 1, hw)
    out = pl.pallas_call(
        _copy_mask_kernel,
        grid=(b, nmask),
        in_specs=[
            pl.BlockSpec((1, cb, hw), lambda i, j: (i, j, 0)),
            pl.BlockSpec((1, 1, 1, hw), lambda i, j: (j, i, 0, 0)),
        ],
        out_specs=pl.BlockSpec((1, cb, hw), lambda i, j: (i, j, 0)),
        out_shape=jax.ShapeDtypeStruct((b, c, hw), x.dtype),
    )(x, nz)
    return out.reshape(b, c, h, w)
